# baseline (speedup 1.0000x reference)
import jax
import jax.numpy as jnp
import numpy as np
from jax import lax
from jax.experimental import pallas as pl
from jax.experimental.pallas import tpu as pltpu
from jax.experimental.pallas import tpu_sc as plsc

N_NODES = 10000
N_EDGES = 160000
F = 128
NBASIS = 4
YW = NBASIS * F

SCH = 32
GCH = 64
NGCH = N_EDGES // GCH
NSCH = N_EDGES // SCH
NC, NS = 2, 16
NW = NC * NS
NPAD = 10240
ROWS_PER_TILE = NPAD // NS
SUPER = 16

G_BASE = NGCH // NW
G_REM = NGCH % NW

_MM_BLOCK = 2000

_PERM = np.empty(YW, np.int32)
for _g in range(YW // 32):
    for _t in range(16):
        _PERM[_g * 32 + 2 * _t] = _g * 32 + _t
        _PERM[_g * 32 + 2 * _t + 1] = _g * 32 + 16 + _t


def _mm_body(x_ref, w_ref, y_ref):
    y_ref[...] = jnp.dot(x_ref[...], w_ref[...],
                         preferred_element_type=jnp.float32
                         ).astype(jnp.bfloat16)


def _add_body(p_ref, o_ref):
    o_ref[...] = p_ref[0] + p_ref[1]


def _sc_body(y_hbm, jg_hbm, i2_hbm, pq_hbm, part_hbm,
             js_v, is_v, pqs_v, rows_v, msg_v, acc_sh,
             gsem0, gsem1, psem, ssem0, ssem1):
    c = lax.axis_index("c")
    s = lax.axis_index("s")
    w = s * NC + c

    for mh in range(2):
        @pl.loop(0, SCH)
        def _zero_rows(r):
            for fb in range(F // 16):
                msg_v[mh, r, pl.ds(fb * 16, 16)] = jnp.zeros((16,),
                                                             jnp.float32)

    for t in range(ROWS_PER_TILE // SCH):
        pltpu.sync_copy(msg_v.at[t % 2],
                        acc_sh.at[pl.ds(s * ROWS_PER_TILE + t * SCH, SCH)])
    plsc.subcore_barrier()

    n_gch = G_BASE + jnp.where(w < G_REM, 1, 0)
    base_g = w * G_BASE + jnp.minimum(w, G_REM)

    gsems = (gsem0, gsem1)
    ssems = (ssem0, ssem1)
    rows_bufs = (rows_v.at[0], rows_v.at[1])
    n_super = (n_gch + SUPER - 1) // SUPER

    def issue_pk(sp, sb):
        pltpu.async_copy(jg_hbm.at[pl.ds(base_g + sp * SUPER, SUPER)],
                         js_v.at[sb], psem)
        sbase = 2 * (base_g + sp * SUPER)
        pltpu.async_copy(i2_hbm.at[pl.ds(sbase, 2 * SUPER)], is_v.at[sb],
                         psem)
        pltpu.async_copy(pq_hbm.at[pl.ds(sbase, 2 * SUPER)], pqs_v.at[sb],
                         psem)

    def wait_pk(sb):
        pltpu.make_async_copy(jg_hbm.at[pl.ds(0, SUPER)], js_v.at[sb],
                              psem).wait()
        pltpu.make_async_copy(i2_hbm.at[pl.ds(0, 2 * SUPER)], is_v.at[sb],
                              psem).wait()
        pltpu.make_async_copy(pq_hbm.at[pl.ds(0, 2 * SUPER)], pqs_v.at[sb],
                              psem).wait()

    def issue_gather(sb, u, b):
        pltpu.async_copy(y_hbm.at[js_v.at[sb, u]], rows_bufs[b], gsems[b])

    def wait_gather(sb, u, b):
        pltpu.make_async_copy(y_hbm.at[js_v.at[sb, u]], rows_bufs[b],
                              gsems[b]).wait()

    def wait_scatter(mh):
        pltpu.make_async_copy(msg_v.at[mh], acc_sh.at[is_v.at[0, 0]],
                              ssems[mh]).wait()

    def compute_sub(sb, su, rows, off, mh):
        wait_scatter(mh)
        for g in range(SCH // 16):
            praw = pqs_v[sb, su, pl.ds(g * 16, 16)]
            pvec = plsc.bitcast(praw << 16, jnp.float32)
            qvec = plsc.bitcast(praw & jnp.int32(-65536), jnp.float32)
            one = jnp.float32(1.0)
            half = jnp.float32(0.5)
            zero = jnp.float32(0.0)
            u0 = jnp.maximum(zero, one - half * jnp.abs(pvec + one))
            u1 = jnp.maximum(zero, one - half * jnp.abs(pvec - one))
            v0 = jnp.maximum(zero, one - half * jnp.abs(qvec + one))
            v1 = jnp.maximum(zero, one - half * jnp.abs(qvec - one))
            c00 = u0 * v0
            c01 = u0 * v1
            c10 = u1 * v0
            c11 = u1 * v1

            @pl.loop(0, 16)
            def _edge(k):
                kvec = jnp.full((16,), k, jnp.int32)
                b0 = c00[kvec]
                b1 = c01[kvec]
                b2 = c10[kvec]
                b3 = c11[kvec]
                bc = (b0, b1, b2, b3)
                e = g * 16 + k
                for fb in range(F // 32):
                    o = fb * 32
                    ta = []
                    tb = []
                    for ab in range(NBASIS):
                        raw = rows[off + e, pl.ds(ab * (F // 2) + fb * 16, 16)]
                        ra = plsc.bitcast(raw << 16, jnp.float32)
                        rb = plsc.bitcast(raw & jnp.int32(-65536), jnp.float32)
                        ta.append(ra * bc[ab])
                        tb.append(rb * bc[ab])
                    msg_v[mh, e, pl.ds(o, 16)] = (ta[0] + ta[1]) + (ta[2] + ta[3])
                    msg_v[mh, e, pl.ds(o + 16, 16)] = (tb[0] + tb[1]) + (tb[2] + tb[3])

        pltpu.async_copy(msg_v.at[mh], acc_sh.at[is_v.at[sb, su]],
                         ssems[mh], add=True)

    issue_pk(0, 0)
    wait_pk(0)
    for mh in range(2):
        pltpu.async_copy(msg_v.at[mh], acc_sh.at[is_v.at[0, 0]],
                         ssems[mh], add=True)
    issue_gather(0, 0, 0)

    @pl.loop(0, n_super)
    def _super(sp):
        sb = sp & 1
        ngc_s = jnp.minimum(n_gch - sp * SUPER, SUPER)

        @pl.loop(0, (ngc_s + 1) // 2)
        def _pair(gp):
            t0 = 2 * gp

            @pl.when(t0 + 1 < ngc_s)
            def _():
                issue_gather(sb, t0 + 1, 1)

            wait_gather(sb, t0, 0)
            compute_sub(sb, 2 * t0, rows_bufs[0], 0, 0)
            compute_sub(sb, 2 * t0 + 1, rows_bufs[0], SCH, 1)

            @pl.when(t0 + 2 < ngc_s)
            def _():
                issue_gather(sb, t0 + 2, 0)

            @pl.when((gp == 0) & ((sp + 1) * SUPER < n_gch))
            def _():
                issue_pk(sp + 1, 1 - sb)

            @pl.when(t0 + 1 < ngc_s)
            def _():
                wait_gather(sb, t0 + 1, 1)
                compute_sub(sb, 2 * t0 + 2, rows_bufs[1], 0, 0)
                compute_sub(sb, 2 * t0 + 3, rows_bufs[1], SCH, 1)

        @pl.when((sp + 1) * SUPER < n_gch)
        def _():
            wait_pk(1 - sb)
            issue_gather(1 - sb, 0, 0)

    wait_scatter(0)
    wait_scatter(1)
    plsc.subcore_barrier()
    pltpu.sync_copy(acc_sh.at[pl.ds(s * ROWS_PER_TILE, ROWS_PER_TILE)],
                    part_hbm.at[c, pl.ds(s * ROWS_PER_TILE, ROWS_PER_TILE)])


_sc_call = pl.kernel(
    _sc_body,
    out_type=jax.ShapeDtypeStruct((NC, NPAD, F), jnp.float32),
    mesh=plsc.VectorSubcoreMesh(core_axis_name="c", subcore_axis_name="s"),
    compiler_params=pltpu.CompilerParams(use_tc_tiling_on_sc=False,
                                         needs_layout_passes=False),
    scratch_types=[
        pltpu.VMEM((2, SUPER, GCH), jnp.int32),
        pltpu.VMEM((2, 2 * SUPER, SCH), jnp.int32),
        pltpu.VMEM((2, 2 * SUPER, SCH), jnp.int32),
        pltpu.VMEM((2, GCH, YW // 2), jnp.int32),
        pltpu.VMEM((2, SCH, F), jnp.float32),
        pltpu.VMEM_SHARED((NPAD, F), jnp.float32),
        pltpu.SemaphoreType.DMA,
        pltpu.SemaphoreType.DMA,
        pltpu.SemaphoreType.DMA,
        pltpu.SemaphoreType.DMA,
        pltpu.SemaphoreType.DMA,
    ],
)


@jax.jit
def kernel(x, edge_index, edge_attr, weight):
    w_flat = weight.transpose(2, 0, 1, 3).reshape(F, YW)[:, _PERM]
    grid = N_NODES // _MM_BLOCK
    y = pl.pallas_call(
        _mm_body,
        grid=(grid,),
        in_specs=[
            pl.BlockSpec((_MM_BLOCK, F), lambda i: (i, 0)),
            pl.BlockSpec((F, YW), lambda i: (0, 0)),
        ],
        out_specs=pl.BlockSpec((_MM_BLOCK, YW), lambda i: (i, 0)),
        out_shape=jax.ShapeDtypeStruct((N_NODES, YW), jnp.bfloat16),
    )(x, w_flat)

    jg = edge_index[1].reshape(NGCH, GCH)
    i2 = edge_index[0].reshape(NSCH, SCH)
    pq = lax.bitcast_convert_type(edge_attr.astype(jnp.bfloat16),
                                  jnp.int32).reshape(NSCH, SCH)

    y_i32 = lax.bitcast_convert_type(y.reshape(N_NODES, YW // 2, 2),
                                     jnp.int32)
    partials = _sc_call(y_i32, jg, i2, pq)

    out = pl.pallas_call(
        _add_body,
        grid=(grid,),
        in_specs=[pl.BlockSpec((NC, _MM_BLOCK, F), lambda i: (0, i, 0))],
        out_specs=pl.BlockSpec((_MM_BLOCK, F), lambda i: (i, 0)),
        out_shape=jax.ShapeDtypeStruct((N_NODES, F), jnp.float32),
    )(partials)
    return out


# speedup vs baseline: 1.1970x; 1.1970x over previous
import jax
import jax.numpy as jnp
import numpy as np
from jax import lax
from jax.experimental import pallas as pl
from jax.experimental.pallas import tpu as pltpu
from jax.experimental.pallas import tpu_sc as plsc

N_NODES = 10000
N_EDGES = 160000
F = 128
NBASIS = 4
YW = NBASIS * F

SCH = 32
GCH = 64
NGCH = N_EDGES // GCH
NSCH = N_EDGES // SCH
NC, NS = 2, 16
NW = NC * NS
NPAD = 10240
ROWS_PER_TILE = NPAD // NS
SUPER = 12

G_BASE = NGCH // NW
G_REM = NGCH % NW

_MM_BLOCK = 2000

_PERM = np.empty(YW, np.int32)
for _g in range(YW // 32):
    for _t in range(16):
        _PERM[_g * 32 + 2 * _t] = _g * 32 + _t
        _PERM[_g * 32 + 2 * _t + 1] = _g * 32 + 16 + _t


def _mm_body(x_ref, w_ref, y_ref):
    y_ref[...] = jnp.dot(x_ref[...], w_ref[...],
                         preferred_element_type=jnp.float32
                         ).astype(jnp.bfloat16)


def _add_body(p_ref, o_ref):
    o_ref[...] = p_ref[0] + p_ref[1]


def _sc_body(y_hbm, jg_hbm, i2_hbm, ea_hbm, part_hbm,
             js_v, is_v, eas_v, rows_v, msg_v, acc_sh,
             gsem0, gsem1, psem, ssem0, ssem1):
    c = lax.axis_index("c")
    s = lax.axis_index("s")
    w = s * NC + c

    for mh in range(2):
        @pl.loop(0, SCH)
        def _zero_rows(r):
            for fb in range(F // 16):
                msg_v[mh, r, pl.ds(fb * 16, 16)] = jnp.zeros((16,),
                                                             jnp.float32)

    for t in range(ROWS_PER_TILE // SCH):
        pltpu.sync_copy(msg_v.at[t % 2],
                        acc_sh.at[pl.ds(s * ROWS_PER_TILE + t * SCH, SCH)])
    plsc.subcore_barrier()

    n_gch = G_BASE + jnp.where(w < G_REM, 1, 0)
    base_g = w * G_BASE + jnp.minimum(w, G_REM)

    gsems = (gsem0, gsem1)
    ssems = (ssem0, ssem1)
    rows_bufs = (rows_v.at[0], rows_v.at[1])
    n_super = (n_gch + SUPER - 1) // SUPER

    def issue_pk(sp, sb):
        pltpu.async_copy(jg_hbm.at[pl.ds(base_g + sp * SUPER, SUPER)],
                         js_v.at[sb], psem)
        sbase = 2 * (base_g + sp * SUPER)
        pltpu.async_copy(i2_hbm.at[pl.ds(sbase, 2 * SUPER)], is_v.at[sb],
                         psem)
        pltpu.async_copy(ea_hbm.at[pl.ds(sbase, 2 * SUPER)], eas_v.at[sb],
                         psem)

    def wait_pk(sb):
        pltpu.make_async_copy(jg_hbm.at[pl.ds(0, SUPER)], js_v.at[sb],
                              psem).wait()
        pltpu.make_async_copy(i2_hbm.at[pl.ds(0, 2 * SUPER)], is_v.at[sb],
                              psem).wait()
        pltpu.make_async_copy(ea_hbm.at[pl.ds(0, 2 * SUPER)], eas_v.at[sb],
                              psem).wait()

    def issue_gather(sb, u, b):
        pltpu.async_copy(y_hbm.at[js_v.at[sb, u]], rows_bufs[b], gsems[b])

    def wait_gather(sb, u, b):
        pltpu.make_async_copy(y_hbm.at[js_v.at[sb, u]], rows_bufs[b],
                              gsems[b]).wait()

    def wait_scatter(mh):
        pltpu.make_async_copy(msg_v.at[mh], acc_sh.at[is_v.at[0, 0]],
                              ssems[mh]).wait()

    def compute_sub(sb, su, rows, off, mh):
        wait_scatter(mh)
        for g in range(SCH // 16):
            lo = eas_v[sb, su, pl.ds(g * 32, 16)]
            hi = eas_v[sb, su, pl.ds(g * 32 + 16, 16)]
            lane = lax.iota(jnp.int32, 16)
            pidx = (2 * lane) & 15
            qidx = (2 * lane + 1) & 15
            in_lo = lane < 8
            pvec = jnp.where(in_lo, lo[pidx], hi[pidx])
            qvec = jnp.where(in_lo, lo[qidx], hi[qidx])
            one = jnp.float32(1.0)
            half = jnp.float32(0.5)
            zero = jnp.float32(0.0)
            u0 = jnp.maximum(zero, one - half * jnp.abs(pvec + one))
            u1 = jnp.maximum(zero, one - half * jnp.abs(pvec - one))
            v0 = jnp.maximum(zero, one - half * jnp.abs(qvec + one))
            v1 = jnp.maximum(zero, one - half * jnp.abs(qvec - one))
            c00 = u0 * v0
            c01 = u0 * v1
            c10 = u1 * v0
            c11 = u1 * v1

            @pl.loop(0, 16)
            def _edge(k):
                kvec = jnp.full((16,), k, jnp.int32)
                b0 = c00[kvec]
                b1 = c01[kvec]
                b2 = c10[kvec]
                b3 = c11[kvec]
                bc = (b0, b1, b2, b3)
                e = g * 16 + k
                for fb in range(F // 32):
                    o = fb * 32
                    ta = []
                    tb = []
                    for ab in range(NBASIS):
                        raw = rows[off + e, pl.ds(ab * F + o, 32)]
                        ra, rb = plsc.unpack(raw,
                                             format=plsc.PackFormat.INTERLEAVED)
                        ta.append(ra * bc[ab])
                        tb.append(rb * bc[ab])
                    msg_v[mh, e, pl.ds(o, 16)] = (ta[0] + ta[1]) + (ta[2] + ta[3])
                    msg_v[mh, e, pl.ds(o + 16, 16)] = (tb[0] + tb[1]) + (tb[2] + tb[3])

        pltpu.async_copy(msg_v.at[mh], acc_sh.at[is_v.at[sb, su]],
                         ssems[mh], add=True)

    issue_pk(0, 0)
    wait_pk(0)
    for mh in range(2):
        pltpu.async_copy(msg_v.at[mh], acc_sh.at[is_v.at[0, 0]],
                         ssems[mh], add=True)
    issue_gather(0, 0, 0)

    @pl.loop(0, n_super)
    def _super(sp):
        sb = sp & 1
        ngc_s = jnp.minimum(n_gch - sp * SUPER, SUPER)

        @pl.loop(0, (ngc_s + 1) // 2)
        def _pair(gp):
            t0 = 2 * gp

            @pl.when(t0 + 1 < ngc_s)
            def _():
                issue_gather(sb, t0 + 1, 1)

            wait_gather(sb, t0, 0)
            compute_sub(sb, 2 * t0, rows_bufs[0], 0, 0)
            compute_sub(sb, 2 * t0 + 1, rows_bufs[0], SCH, 1)

            @pl.when(t0 + 2 < ngc_s)
            def _():
                issue_gather(sb, t0 + 2, 0)

            @pl.when((gp == 0) & ((sp + 1) * SUPER < n_gch))
            def _():
                issue_pk(sp + 1, 1 - sb)

            @pl.when(t0 + 1 < ngc_s)
            def _():
                wait_gather(sb, t0 + 1, 1)
                compute_sub(sb, 2 * t0 + 2, rows_bufs[1], 0, 0)
                compute_sub(sb, 2 * t0 + 3, rows_bufs[1], SCH, 1)

        @pl.when((sp + 1) * SUPER < n_gch)
        def _():
            wait_pk(1 - sb)
            issue_gather(1 - sb, 0, 0)

    wait_scatter(0)
    wait_scatter(1)
    plsc.subcore_barrier()
    pltpu.sync_copy(acc_sh.at[pl.ds(s * ROWS_PER_TILE, ROWS_PER_TILE)],
                    part_hbm.at[c, pl.ds(s * ROWS_PER_TILE, ROWS_PER_TILE)])


_sc_call = pl.kernel(
    _sc_body,
    out_type=jax.ShapeDtypeStruct((NC, NPAD, F), jnp.float32),
    mesh=plsc.VectorSubcoreMesh(core_axis_name="c", subcore_axis_name="s"),
    compiler_params=pltpu.CompilerParams(use_tc_tiling_on_sc=False,
                                         needs_layout_passes=False),
    scratch_types=[
        pltpu.VMEM((2, SUPER, GCH), jnp.int32),
        pltpu.VMEM((2, 2 * SUPER, SCH), jnp.int32),
        pltpu.VMEM((2, 2 * SUPER, 2 * SCH), jnp.float32),
        pltpu.VMEM((2, GCH, YW), jnp.bfloat16),
        pltpu.VMEM((2, SCH, F), jnp.float32),
        pltpu.VMEM_SHARED((NPAD, F), jnp.float32),
        pltpu.SemaphoreType.DMA,
        pltpu.SemaphoreType.DMA,
        pltpu.SemaphoreType.DMA,
        pltpu.SemaphoreType.DMA,
        pltpu.SemaphoreType.DMA,
    ],
)


@jax.jit
def kernel(x, edge_index, edge_attr, weight):
    w_flat = weight.transpose(2, 0, 1, 3).reshape(F, YW)[:, _PERM]
    x16 = x.astype(jnp.bfloat16)
    w16 = w_flat.astype(jnp.bfloat16)
    grid = N_NODES // _MM_BLOCK
    y = pl.pallas_call(
        _mm_body,
        grid=(grid,),
        in_specs=[
            pl.BlockSpec((_MM_BLOCK, F), lambda i: (i, 0)),
            pl.BlockSpec((F, YW), lambda i: (0, 0)),
        ],
        out_specs=pl.BlockSpec((_MM_BLOCK, YW), lambda i: (i, 0)),
        out_shape=jax.ShapeDtypeStruct((N_NODES, YW), jnp.bfloat16),
    )(x16, w16)

    jg = edge_index[1].reshape(NGCH, GCH)
    i2 = edge_index[0].reshape(NSCH, SCH)
    ea2 = edge_attr.reshape(NSCH, 2 * SCH)

    partials = _sc_call(y, jg, i2, ea2)

    out = pl.pallas_call(
        _add_body,
        grid=(grid,),
        in_specs=[pl.BlockSpec((NC, _MM_BLOCK, F), lambda i: (0, i, 0))],
        out_specs=pl.BlockSpec((_MM_BLOCK, F), lambda i: (i, 0)),
        out_shape=jax.ShapeDtypeStruct((N_NODES, F), jnp.float32),
    )(partials)
    return out


# speedup vs baseline: 1.5444x; 1.2902x over previous
import jax
import jax.numpy as jnp
import numpy as np
from jax import lax
from jax.experimental import pallas as pl
from jax.experimental.pallas import tpu as pltpu
from jax.experimental.pallas import tpu_sc as plsc

N_NODES = 10000
N_EDGES = 160000
F = 128
NBASIS = 4
YW = NBASIS * F

SCH = 32
GCH = 64
NGCH = N_EDGES // GCH
NSCH = N_EDGES // SCH
NC, NS = 2, 16
NW = NC * NS
NPAD = 10240
ROWS_PER_TILE = NPAD // NS
SUPER = 16

G_BASE = NGCH // NW
G_REM = NGCH % NW

_MM_BLOCK = 2000

_PERM = np.empty(YW, np.int32)
for _g in range(YW // 32):
    for _t in range(16):
        _PERM[_g * 32 + 2 * _t] = _g * 32 + _t
        _PERM[_g * 32 + 2 * _t + 1] = _g * 32 + 16 + _t


def _mm_body(x_ref, w_ref, y_ref):
    y_ref[...] = jnp.dot(x_ref[...], w_ref[...],
                         preferred_element_type=jnp.float32
                         ).astype(jnp.bfloat16)


def _add_body(p_ref, o_ref):
    o_ref[...] = p_ref[0] + p_ref[1]


def _sc_body(y_hbm, jg_hbm, i2_hbm, pq_hbm, part_hbm,
             js_v, is_v, pqs_v, rows_v, msg_v, acc_sh,
             gsem0, gsem1, psem, ssem0, ssem1):
    c = lax.axis_index("c")
    s = lax.axis_index("s")
    w = s * NC + c

    for mh in range(2):
        @pl.loop(0, SCH)
        def _zero_rows(r):
            for fb in range(F // 16):
                msg_v[mh, r, pl.ds(fb * 16, 16)] = jnp.zeros((16,),
                                                             jnp.float32)

    for t in range(ROWS_PER_TILE // SCH):
        pltpu.sync_copy(msg_v.at[t % 2],
                        acc_sh.at[pl.ds(s * ROWS_PER_TILE + t * SCH, SCH)])
    plsc.subcore_barrier()

    n_gch = G_BASE + jnp.where(w < G_REM, 1, 0)
    base_g = w * G_BASE + jnp.minimum(w, G_REM)

    gsems = (gsem0, gsem1)
    ssems = (ssem0, ssem1)
    rows_bufs = (rows_v.at[0], rows_v.at[1])
    n_super = (n_gch + SUPER - 1) // SUPER

    def issue_pk(sp, sb):
        pltpu.async_copy(jg_hbm.at[pl.ds(base_g + sp * SUPER, SUPER)],
                         js_v.at[sb], psem)
        sbase = 2 * (base_g + sp * SUPER)
        pltpu.async_copy(i2_hbm.at[pl.ds(sbase, 2 * SUPER)], is_v.at[sb],
                         psem)
        pltpu.async_copy(pq_hbm.at[pl.ds(sbase, 2 * SUPER)], pqs_v.at[sb],
                         psem)

    def wait_pk(sb):
        pltpu.make_async_copy(jg_hbm.at[pl.ds(0, SUPER)], js_v.at[sb],
                              psem).wait()
        pltpu.make_async_copy(i2_hbm.at[pl.ds(0, 2 * SUPER)], is_v.at[sb],
                              psem).wait()
        pltpu.make_async_copy(pq_hbm.at[pl.ds(0, 2 * SUPER)], pqs_v.at[sb],
                              psem).wait()

    def issue_gather(sb, u, b):
        pltpu.async_copy(y_hbm.at[js_v.at[sb, u]], rows_bufs[b], gsems[b])

    def wait_gather(sb, u, b):
        pltpu.make_async_copy(y_hbm.at[js_v.at[sb, u]], rows_bufs[b],
                              gsems[b]).wait()

    def wait_scatter(mh):
        pltpu.make_async_copy(msg_v.at[mh], acc_sh.at[is_v.at[0, 0]],
                              ssems[mh]).wait()

    def compute_sub(sb, su, rows, off, mh):
        wait_scatter(mh)
        for g in range(SCH // 16):
            pq = plsc.bitcast(pqs_v[sb, su, pl.ds(g * 16, 16)], jnp.bfloat16)
            pvec, qvec = plsc.unpack(pq, format=plsc.PackFormat.INTERLEAVED)
            one = jnp.float32(1.0)
            half = jnp.float32(0.5)
            zero = jnp.float32(0.0)
            u0 = jnp.maximum(zero, one - half * jnp.abs(pvec + one))
            u1 = jnp.maximum(zero, one - half * jnp.abs(pvec - one))
            v0 = jnp.maximum(zero, one - half * jnp.abs(qvec + one))
            v1 = jnp.maximum(zero, one - half * jnp.abs(qvec - one))
            c00 = u0 * v0
            c01 = u0 * v1
            c10 = u1 * v0
            c11 = u1 * v1

            @pl.loop(0, 16)
            def _edge(k):
                kvec = jnp.full((16,), k, jnp.int32)
                b0 = c00[kvec]
                b1 = c01[kvec]
                b2 = c10[kvec]
                b3 = c11[kvec]
                bc = (b0, b1, b2, b3)
                e = g * 16 + k
                for fb in range(F // 32):
                    o = fb * 32
                    ta = []
                    tb = []
                    for ab in range(NBASIS):
                        raw = rows[off + e, pl.ds(ab * F + o, 32)]
                        ra, rb = plsc.unpack(raw,
                                             format=plsc.PackFormat.INTERLEAVED)
                        ta.append(ra * bc[ab])
                        tb.append(rb * bc[ab])
                    msg_v[mh, e, pl.ds(o, 16)] = (ta[0] + ta[1]) + (ta[2] + ta[3])
                    msg_v[mh, e, pl.ds(o + 16, 16)] = (tb[0] + tb[1]) + (tb[2] + tb[3])

        pltpu.async_copy(msg_v.at[mh], acc_sh.at[is_v.at[sb, su]],
                         ssems[mh], add=True)

    issue_pk(0, 0)
    wait_pk(0)
    for mh in range(2):
        pltpu.async_copy(msg_v.at[mh], acc_sh.at[is_v.at[0, 0]],
                         ssems[mh], add=True)
    issue_gather(0, 0, 0)

    @pl.loop(0, n_super)
    def _super(sp):
        sb = sp & 1
        ngc_s = jnp.minimum(n_gch - sp * SUPER, SUPER)

        @pl.loop(0, (ngc_s + 1) // 2)
        def _pair(gp):
            t0 = 2 * gp

            @pl.when(t0 + 1 < ngc_s)
            def _():
                issue_gather(sb, t0 + 1, 1)

            wait_gather(sb, t0, 0)
            compute_sub(sb, 2 * t0, rows_bufs[0], 0, 0)
            compute_sub(sb, 2 * t0 + 1, rows_bufs[0], SCH, 1)

            @pl.when(t0 + 2 < ngc_s)
            def _():
                issue_gather(sb, t0 + 2, 0)

            @pl.when((gp == 0) & ((sp + 1) * SUPER < n_gch))
            def _():
                issue_pk(sp + 1, 1 - sb)

            @pl.when(t0 + 1 < ngc_s)
            def _():
                wait_gather(sb, t0 + 1, 1)
                compute_sub(sb, 2 * t0 + 2, rows_bufs[1], 0, 0)
                compute_sub(sb, 2 * t0 + 3, rows_bufs[1], SCH, 1)

        @pl.when((sp + 1) * SUPER < n_gch)
        def _():
            wait_pk(1 - sb)
            issue_gather(1 - sb, 0, 0)

    wait_scatter(0)
    wait_scatter(1)
    plsc.subcore_barrier()
    pltpu.sync_copy(acc_sh.at[pl.ds(s * ROWS_PER_TILE, ROWS_PER_TILE)],
                    part_hbm.at[c, pl.ds(s * ROWS_PER_TILE, ROWS_PER_TILE)])


_sc_call = pl.kernel(
    _sc_body,
    out_type=jax.ShapeDtypeStruct((NC, NPAD, F), jnp.float32),
    mesh=plsc.VectorSubcoreMesh(core_axis_name="c", subcore_axis_name="s"),
    compiler_params=pltpu.CompilerParams(use_tc_tiling_on_sc=False,
                                         needs_layout_passes=False),
    scratch_types=[
        pltpu.VMEM((2, SUPER, GCH), jnp.int32),
        pltpu.VMEM((2, 2 * SUPER, SCH), jnp.int32),
        pltpu.VMEM((2, 2 * SUPER, SCH), jnp.int32),
        pltpu.VMEM((2, GCH, YW), jnp.bfloat16),
        pltpu.VMEM((2, SCH, F), jnp.float32),
        pltpu.VMEM_SHARED((NPAD, F), jnp.float32),
        pltpu.SemaphoreType.DMA,
        pltpu.SemaphoreType.DMA,
        pltpu.SemaphoreType.DMA,
        pltpu.SemaphoreType.DMA,
        pltpu.SemaphoreType.DMA,
    ],
)


@jax.jit
def kernel(x, edge_index, edge_attr, weight):
    w_flat = weight.transpose(2, 0, 1, 3).reshape(F, YW)[:, _PERM]
    x16 = x.astype(jnp.bfloat16)
    w16 = w_flat.astype(jnp.bfloat16)
    grid = N_NODES // _MM_BLOCK
    y = pl.pallas_call(
        _mm_body,
        grid=(grid,),
        in_specs=[
            pl.BlockSpec((_MM_BLOCK, F), lambda i: (i, 0)),
            pl.BlockSpec((F, YW), lambda i: (0, 0)),
        ],
        out_specs=pl.BlockSpec((_MM_BLOCK, YW), lambda i: (i, 0)),
        out_shape=jax.ShapeDtypeStruct((N_NODES, YW), jnp.bfloat16),
    )(x16, w16)

    jg = edge_index[1].reshape(NGCH, GCH)
    i2 = edge_index[0].reshape(NSCH, SCH)
    pq = lax.bitcast_convert_type(edge_attr.astype(jnp.bfloat16),
                                  jnp.int32).reshape(NSCH, SCH)

    partials = _sc_call(y, jg, i2, pq)

    out = pl.pallas_call(
        _add_body,
        grid=(grid,),
        in_specs=[pl.BlockSpec((NC, _MM_BLOCK, F), lambda i: (0, i, 0))],
        out_specs=pl.BlockSpec((_MM_BLOCK, F), lambda i: (i, 0)),
        out_shape=jax.ShapeDtypeStruct((N_NODES, F), jnp.float32),
    )(partials)
    return out


# speedup vs baseline: 1.6757x; 1.0850x over previous
import jax
import jax.numpy as jnp
import numpy as np
from jax import lax
from jax.experimental import pallas as pl
from jax.experimental.pallas import tpu as pltpu
from jax.experimental.pallas import tpu_sc as plsc

N_NODES = 10000
N_EDGES = 160000
F = 128
NBASIS = 4
YW = NBASIS * F

SCH = 32
GCH = 64
NGCH = N_EDGES // GCH
NSCH = N_EDGES // SCH
NC, NS = 2, 16
NW = NC * NS
NPAD = 10240
ROWS_PER_TILE = NPAD // NS
SUPER = 16

G_BASE = NGCH // NW
G_REM = NGCH % NW

_MM_BLOCK = 2000

_PERM = np.empty(YW, np.int32)
for _g in range(YW // 32):
    for _t in range(16):
        _PERM[_g * 32 + 2 * _t] = _g * 32 + _t
        _PERM[_g * 32 + 2 * _t + 1] = _g * 32 + 16 + _t


def _mm_body(x_ref, w_ref, y_ref):
    y_ref[...] = jnp.dot(x_ref[...], w_ref[...],
                         preferred_element_type=jnp.float32
                         ).astype(jnp.bfloat16)


def _add_body(p_ref, o_ref):
    o_ref[...] = p_ref[0] + p_ref[1]


def _sc_body(y_hbm, jg_hbm, i2_hbm, pq_hbm, part_hbm,
             js_v, is_v, pqs_v, rows_v, msg_v, acc_sh,
             gsem0, gsem1, psem, ssem0, ssem1):
    c = lax.axis_index("c")
    s = lax.axis_index("s")
    w = s * NC + c

    for mh in range(2):
        @pl.loop(0, SCH)
        def _zero_rows(r):
            for fb in range(F // 16):
                msg_v[mh, r, pl.ds(fb * 16, 16)] = jnp.zeros((16,),
                                                             jnp.float32)

    for t in range(ROWS_PER_TILE // SCH):
        pltpu.sync_copy(msg_v.at[t % 2],
                        acc_sh.at[pl.ds(s * ROWS_PER_TILE + t * SCH, SCH)])
    plsc.subcore_barrier()

    n_gch = G_BASE + jnp.where(w < G_REM, 1, 0)
    base_g = w * G_BASE + jnp.minimum(w, G_REM)

    gsems = (gsem0, gsem1)
    ssems = (ssem0, ssem1)
    rows_bufs = (rows_v.at[0], rows_v.at[1])
    n_super = (n_gch + SUPER - 1) // SUPER

    def issue_pk(sp, sb):
        pltpu.async_copy(jg_hbm.at[pl.ds(base_g + sp * SUPER, SUPER)],
                         js_v.at[sb], psem)
        sbase = 2 * (base_g + sp * SUPER)
        pltpu.async_copy(i2_hbm.at[pl.ds(sbase, 2 * SUPER)], is_v.at[sb],
                         psem)
        pltpu.async_copy(pq_hbm.at[pl.ds(sbase, 2 * SUPER)], pqs_v.at[sb],
                         psem)

    def wait_pk(sb):
        pltpu.make_async_copy(jg_hbm.at[pl.ds(0, SUPER)], js_v.at[sb],
                              psem).wait()
        pltpu.make_async_copy(i2_hbm.at[pl.ds(0, 2 * SUPER)], is_v.at[sb],
                              psem).wait()
        pltpu.make_async_copy(pq_hbm.at[pl.ds(0, 2 * SUPER)], pqs_v.at[sb],
                              psem).wait()

    def issue_gather(sb, u, b):
        pltpu.async_copy(y_hbm.at[js_v.at[sb, u]], rows_bufs[b], gsems[b])

    def wait_gather(sb, u, b):
        pltpu.make_async_copy(y_hbm.at[js_v.at[sb, u]], rows_bufs[b],
                              gsems[b]).wait()

    def wait_scatter(mh):
        pltpu.make_async_copy(msg_v.at[mh], acc_sh.at[is_v.at[0, 0]],
                              ssems[mh]).wait()

    def compute_sub(sb, su, rows, off, mh):
        wait_scatter(mh)
        for g in range(SCH // 16):
            pq = plsc.bitcast(pqs_v[sb, su, pl.ds(g * 16, 16)], jnp.bfloat16)
            pvec, qvec = plsc.unpack(pq, format=plsc.PackFormat.INTERLEAVED)
            pqvec = pvec * qvec

            @pl.loop(0, 16)
            def _edge(k):
                kvec = jnp.full((16,), k, jnp.int32)
                b1 = pvec[kvec]
                b2 = qvec[kvec]
                b3 = pqvec[kvec]
                e = g * 16 + k
                for fb in range(F // 32):
                    o = fb * 32
                    ta = []
                    tb = []
                    for ab in range(NBASIS):
                        raw = rows[off + e, pl.ds(ab * F + o, 32)]
                        ta_ab, tb_ab = plsc.unpack(
                            raw, format=plsc.PackFormat.INTERLEAVED)
                        ta.append(ta_ab)
                        tb.append(tb_ab)
                    msg_v[mh, e, pl.ds(o, 16)] = (
                        (ta[0] + ta[1] * b1) + (ta[2] * b2 + ta[3] * b3))
                    msg_v[mh, e, pl.ds(o + 16, 16)] = (
                        (tb[0] + tb[1] * b1) + (tb[2] * b2 + tb[3] * b3))

        pltpu.async_copy(msg_v.at[mh], acc_sh.at[is_v.at[sb, su]],
                         ssems[mh], add=True)

    issue_pk(0, 0)
    wait_pk(0)
    for mh in range(2):
        pltpu.async_copy(msg_v.at[mh], acc_sh.at[is_v.at[0, 0]],
                         ssems[mh], add=True)
    issue_gather(0, 0, 0)

    @pl.loop(0, n_super)
    def _super(sp):
        sb = sp & 1
        ngc_s = jnp.minimum(n_gch - sp * SUPER, SUPER)

        @pl.loop(0, (ngc_s + 1) // 2)
        def _pair(gp):
            t0 = 2 * gp

            @pl.when(t0 + 1 < ngc_s)
            def _():
                issue_gather(sb, t0 + 1, 1)

            wait_gather(sb, t0, 0)
            compute_sub(sb, 2 * t0, rows_bufs[0], 0, 0)
            compute_sub(sb, 2 * t0 + 1, rows_bufs[0], SCH, 1)

            @pl.when(t0 + 2 < ngc_s)
            def _():
                issue_gather(sb, t0 + 2, 0)

            @pl.when((gp == 0) & ((sp + 1) * SUPER < n_gch))
            def _():
                issue_pk(sp + 1, 1 - sb)

            @pl.when(t0 + 1 < ngc_s)
            def _():
                wait_gather(sb, t0 + 1, 1)
                compute_sub(sb, 2 * t0 + 2, rows_bufs[1], 0, 0)
                compute_sub(sb, 2 * t0 + 3, rows_bufs[1], SCH, 1)

        @pl.when((sp + 1) * SUPER < n_gch)
        def _():
            wait_pk(1 - sb)
            issue_gather(1 - sb, 0, 0)

    wait_scatter(0)
    wait_scatter(1)
    plsc.subcore_barrier()
    pltpu.sync_copy(acc_sh.at[pl.ds(s * ROWS_PER_TILE, ROWS_PER_TILE)],
                    part_hbm.at[c, pl.ds(s * ROWS_PER_TILE, ROWS_PER_TILE)])


_sc_call = pl.kernel(
    _sc_body,
    out_type=jax.ShapeDtypeStruct((NC, NPAD, F), jnp.float32),
    mesh=plsc.VectorSubcoreMesh(core_axis_name="c", subcore_axis_name="s"),
    compiler_params=pltpu.CompilerParams(use_tc_tiling_on_sc=False,
                                         needs_layout_passes=False),
    scratch_types=[
        pltpu.VMEM((2, SUPER, GCH), jnp.int32),
        pltpu.VMEM((2, 2 * SUPER, SCH), jnp.int32),
        pltpu.VMEM((2, 2 * SUPER, SCH), jnp.int32),
        pltpu.VMEM((2, GCH, YW), jnp.bfloat16),
        pltpu.VMEM((2, SCH, F), jnp.float32),
        pltpu.VMEM_SHARED((NPAD, F), jnp.float32),
        pltpu.SemaphoreType.DMA,
        pltpu.SemaphoreType.DMA,
        pltpu.SemaphoreType.DMA,
        pltpu.SemaphoreType.DMA,
        pltpu.SemaphoreType.DMA,
    ],
)


@jax.jit
def kernel(x, edge_index, edge_attr, weight):
    t0 = weight[0, 0] + weight[0, 1] + weight[1, 0] + weight[1, 1]
    tp = weight[1, 0] + weight[1, 1] - weight[0, 0] - weight[0, 1]
    tq = weight[0, 1] + weight[1, 1] - weight[0, 0] - weight[1, 0]
    tpq = weight[0, 0] - weight[0, 1] - weight[1, 0] + weight[1, 1]
    w_flat = (jnp.concatenate([t0, tp, tq, tpq], axis=1) * 0.25)[:, _PERM]
    grid = N_NODES // _MM_BLOCK
    y = pl.pallas_call(
        _mm_body,
        grid=(grid,),
        in_specs=[
            pl.BlockSpec((_MM_BLOCK, F), lambda i: (i, 0)),
            pl.BlockSpec((F, YW), lambda i: (0, 0)),
        ],
        out_specs=pl.BlockSpec((_MM_BLOCK, YW), lambda i: (i, 0)),
        out_shape=jax.ShapeDtypeStruct((N_NODES, YW), jnp.bfloat16),
    )(x, w_flat)

    jg = edge_index[1].reshape(NGCH, GCH)
    i2 = edge_index[0].reshape(NSCH, SCH)
    pq = lax.bitcast_convert_type(edge_attr.astype(jnp.bfloat16),
                                  jnp.int32).reshape(NSCH, SCH)

    partials = _sc_call(y, jg, i2, pq)

    out = pl.pallas_call(
        _add_body,
        grid=(grid,),
        in_specs=[pl.BlockSpec((NC, _MM_BLOCK, F), lambda i: (0, i, 0))],
        out_specs=pl.BlockSpec((_MM_BLOCK, F), lambda i: (i, 0)),
        out_shape=jax.ShapeDtypeStruct((N_NODES, F), jnp.float32),
    )(partials)
    return out
